# baseline (speedup 1.0000x reference)
import functools

import jax
import jax.numpy as jnp
from jax.experimental import pallas as pl
from jax.experimental.pallas import tpu as pltpu

_INV_SQRT2 = 0.7071067811865476
_LN_EPS = 1e-5


def _yc_kernel(a_ref, b_ref, c_ref, out_ref, *, T):
    S = a_ref.shape[0]
    hi = jax.lax.Precision.HIGHEST
    statesT = jnp.broadcast_to(b_ref[...], (S, T))
    P = a_ref[...]
    shift = 1
    while shift < T:
        shifted = jnp.concatenate(
            [jnp.zeros((S, shift), jnp.float32), statesT[:, : T - shift]], axis=1
        )
        statesT = statesT + jax.lax.dot(
            P, shifted, precision=hi, preferred_element_type=jnp.float32
        )
        shift *= 2
        if shift < T:
            P = jax.lax.dot(P, P, precision=hi, preferred_element_type=jnp.float32)
    out_ref[...] = jax.lax.dot_general(
        c_ref[...], statesT, (((0,), (0,)), ((), ())),
        precision=hi, preferred_element_type=jnp.float32,
    )


def _compute_chunk(xv, yc, d, w, bias):
    t = xv * d + yc
    g = 0.5 * t * (1.0 + jax.lax.erf(t * _INV_SQRT2))
    o = xv + g
    mu = jnp.mean(o, axis=0, keepdims=True)
    m2 = jnp.mean(o * o, axis=0, keepdims=True)
    var = m2 - mu * mu
    rs = jax.lax.rsqrt(var + _LN_EPS)
    return (o - mu) * rs * w + bias


def _fused_manual(yc_ref, d_ref, w_ref, b_ref, x_hbm, out_hbm,
                  in_buf, out_buf, in_sem, out_sem, *, N, HT, HB):
    def in_copy(i, slot):
        b = i // HT
        h = i % HT
        return pltpu.make_async_copy(
            x_hbm.at[b, :, pl.ds(h * HB, HB), :], in_buf.at[slot], in_sem.at[slot]
        )

    def out_copy(i, slot):
        b = i // HT
        h = i % HT
        return pltpu.make_async_copy(
            out_buf.at[slot], out_hbm.at[b, :, pl.ds(h * HB, HB), :], out_sem.at[slot]
        )

    yc = yc_ref[...]
    d = d_ref[...]
    w = w_ref[...]
    bias = b_ref[...]

    in_copy(0, 0).start()

    def body(j, carry):
        for k in (0, 1):
            i = 2 * j + k
            nslot = (k + 1) % 2

            @pl.when(i + 1 < N)
            def _():
                in_copy(i + 1, nslot).start()

            in_copy(i, k).wait()

            @pl.when(i >= 2)
            def _():
                out_copy(i - 2, k).wait()

            out_buf[k] = _compute_chunk(in_buf[k], yc, d, w, bias)
            out_copy(i, k).start()
        return carry

    jax.lax.fori_loop(0, N // 2, body, 0)
    out_copy(N - 2, 0).wait()
    out_copy(N - 1, 1).wait()


def kernel(x, A, Bvec, Cmat, D, ln_w, ln_b):
    B, F, H, T = x.shape
    S = A.shape[0]
    HB = 16
    HT = H // HB
    N = B * HT

    yc = pl.pallas_call(
        functools.partial(_yc_kernel, T=T),
        out_shape=jax.ShapeDtypeStruct((F, T), jnp.float32),
        name="ssm_states_yc",
    )(A, Bvec.reshape(S, 1), Cmat)

    yc3 = yc.reshape(F, 1, T)
    d3 = D.reshape(F, 1, 1)
    w3 = ln_w.reshape(F, 1, 1)
    b3 = ln_b.reshape(F, 1, 1)

    vspec = pl.BlockSpec(memory_space=pltpu.VMEM)
    out = pl.pallas_call(
        functools.partial(_fused_manual, N=N, HT=HT, HB=HB),
        in_specs=[vspec, vspec, vspec, vspec,
                  pl.BlockSpec(memory_space=pl.ANY)],
        out_specs=pl.BlockSpec(memory_space=pl.ANY),
        out_shape=jax.ShapeDtypeStruct(x.shape, x.dtype),
        scratch_shapes=[
            pltpu.VMEM((2, F, HB, T), jnp.float32),
            pltpu.VMEM((2, F, HB, T), jnp.float32),
            pltpu.SemaphoreType.DMA((2,)),
            pltpu.SemaphoreType.DMA((2,)),
        ],
        compiler_params=pltpu.CompilerParams(
            vmem_limit_bytes=56 * 1024 * 1024,
        ),
        name="ssm_gelu_ln_manual",
    )(yc3, d3, w3, b3, x)
    return out


# speedup vs baseline: 1.1349x; 1.1349x over previous
import functools

import jax
import jax.numpy as jnp
from jax.experimental import pallas as pl
from jax.experimental.pallas import tpu as pltpu

_INV_SQRT2 = 0.7071067811865476
_LN_EPS = 1e-5


def _yc_kernel(a_ref, b_ref, c_ref, out_ref, *, T):
    S = a_ref.shape[0]
    hi = jax.lax.Precision.HIGHEST
    statesT = jnp.broadcast_to(b_ref[...], (S, T))
    P = a_ref[...]
    shift = 1
    while shift < T:
        shifted = jnp.concatenate(
            [jnp.zeros((S, shift), jnp.float32), statesT[:, : T - shift]], axis=1
        )
        statesT = statesT + jax.lax.dot(
            P, shifted, precision=hi, preferred_element_type=jnp.float32
        )
        shift *= 2
        if shift < T:
            P = jax.lax.dot(P, P, precision=hi, preferred_element_type=jnp.float32)
    yct = jax.lax.dot_general(
        c_ref[...], statesT, (((0,), (0,)), ((), ())),
        precision=hi, preferred_element_type=jnp.float32,
    )
    out_ref[...] = yct.reshape(1, yct.shape[0], 1, T)


def _fused_kernel(x_ref, yc_ref, d_ref, w_ref, bias_ref, out_ref):
    xv = x_ref[...]
    t = xv * d_ref[...] + yc_ref[...]
    g = 0.5 * t * (1.0 + jax.lax.erf(t * _INV_SQRT2))
    o = xv + g
    mu = jnp.mean(o, axis=1, keepdims=True)
    m2 = jnp.mean(o * o, axis=1, keepdims=True)
    var = m2 - mu * mu
    rs = jax.lax.rsqrt(var + _LN_EPS)
    out_ref[...] = (o - mu) * rs * w_ref[...] + bias_ref[...]


def kernel(x, A, Bvec, Cmat, D, ln_w, ln_b):
    B, F, H, T = x.shape
    S = A.shape[0]
    HB = 16

    yc4 = pl.pallas_call(
        functools.partial(_yc_kernel, T=T),
        out_shape=jax.ShapeDtypeStruct((1, F, 1, T), jnp.float32),
        name="ssm_states_yc",
    )(A, Bvec.reshape(S, 1), Cmat)

    d4 = D.reshape(1, F, 1, 1)
    w4 = ln_w.reshape(1, F, 1, 1)
    b4 = ln_b.reshape(1, F, 1, 1)

    HT = H // HB

    const_spec = pl.BlockSpec((1, F, 1, 1), lambda i: (0, 0, 0, 0))
    out = pl.pallas_call(
        _fused_kernel,
        grid=(B * HT,),
        in_specs=[
            pl.BlockSpec((1, F, HB, T), lambda i: (i // HT, 0, i % HT, 0)),
            pl.BlockSpec((1, F, 1, T), lambda i: (0, 0, 0, 0)),
            const_spec,
            const_spec,
            const_spec,
        ],
        out_specs=pl.BlockSpec((1, F, HB, T), lambda i: (i // HT, 0, i % HT, 0)),
        out_shape=jax.ShapeDtypeStruct(x.shape, x.dtype),
        compiler_params=pltpu.CompilerParams(
            dimension_semantics=("arbitrary",),
            vmem_limit_bytes=52 * 1024 * 1024,
        ),
        name="ssm_gelu_ln",
    )(x, yc4, d4, w4, b4)
    return out


# speedup vs baseline: 1.1396x; 1.0041x over previous
import functools

import jax
import jax.numpy as jnp
from jax.experimental import pallas as pl
from jax.experimental.pallas import tpu as pltpu

_INV_SQRT2 = 0.7071067811865476
_LN_EPS = 1e-5


def _yc_kernel(a_ref, b_ref, c_ref, out_ref, *, T):
    S = a_ref.shape[0]
    hi = jax.lax.Precision.HIGHEST
    statesT = jnp.broadcast_to(b_ref[...], (S, T))
    P = a_ref[...]
    shift = 1
    while shift < T:
        shifted = jnp.concatenate(
            [jnp.zeros((S, shift), jnp.float32), statesT[:, : T - shift]], axis=1
        )
        statesT = statesT + jax.lax.dot(
            P, shifted, precision=hi, preferred_element_type=jnp.float32
        )
        shift *= 2
        if shift < T:
            P = jax.lax.dot(P, P, precision=hi, preferred_element_type=jnp.float32)
    yct = jax.lax.dot_general(
        c_ref[...], statesT, (((0,), (0,)), ((), ())),
        precision=hi, preferred_element_type=jnp.float32,
    )
    out_ref[...] = yct.reshape(1, yct.shape[0], 1, T)


def _fused_kernel(x_ref, yc_ref, c_ref, out_ref):
    cv = c_ref[...]
    d = cv[:, :, 0:1, :]
    w = cv[:, :, 1:2, :]
    bias = cv[:, :, 2:3, :]
    xv = x_ref[...]
    t = xv * d + yc_ref[...]
    g = 0.5 * t * (1.0 + jax.lax.erf(t * _INV_SQRT2))
    o = xv + g
    mu = jnp.mean(o, axis=1, keepdims=True)
    m2 = jnp.mean(o * o, axis=1, keepdims=True)
    var = m2 - mu * mu
    rs = jax.lax.rsqrt(var + _LN_EPS)
    out_ref[...] = (o - mu) * rs * w + bias


def kernel(x, A, Bvec, Cmat, D, ln_w, ln_b):
    B, F, H, T = x.shape
    S = A.shape[0]
    HB = 16

    yc4 = pl.pallas_call(
        functools.partial(_yc_kernel, T=T),
        out_shape=jax.ShapeDtypeStruct((1, F, 1, T), jnp.float32),
        name="ssm_states_yc",
    )(A, Bvec.reshape(S, 1), Cmat)

    cpk = jnp.stack([D, ln_w, ln_b], axis=1).reshape(1, F, 3, 1)

    HT = H // HB

    out = pl.pallas_call(
        _fused_kernel,
        grid=(B * HT,),
        in_specs=[
            pl.BlockSpec((1, F, HB, T), lambda i: (i // HT, 0, i % HT, 0)),
            pl.BlockSpec((1, F, 1, T), lambda i: (0, 0, 0, 0)),
            pl.BlockSpec((1, F, 3, 1), lambda i: (0, 0, 0, 0)),
        ],
        out_specs=pl.BlockSpec((1, F, HB, T), lambda i: (i // HT, 0, i % HT, 0)),
        out_shape=jax.ShapeDtypeStruct(x.shape, x.dtype),
        compiler_params=pltpu.CompilerParams(
            dimension_semantics=("arbitrary",),
            vmem_limit_bytes=52 * 1024 * 1024,
        ),
        name="ssm_gelu_ln",
    )(x, yc4, cpk)
    return out
